# baseline (speedup 1.0000x reference)
import jax
import jax.numpy as jnp
from jax import lax
from jax.experimental import pallas as pl
from jax.experimental.pallas import tpu as pltpu
from jax.experimental.pallas import tpu_sc as plsc

N_NODES = 10000
D_HID = 16

NC = 2
NS = 16
NW = NC * NS
B_EDGE = 800
N_PAD = 10240
FRAC0 = 1.0
RING = 4
LOOK = 2


def _cdiv(a, b):
    return (a + b - 1) // b


def _seg_sum(y, ei3, with_cnt):
    ch = ei3.shape[1]
    cht = ch // NS
    k0 = max(1, min(cht, round(cht * FRAC0)))
    k1 = cht - k0
    kmax = max(k0, k1)
    rps = N_PAD // NS

    mesh = plsc.VectorSubcoreMesh(core_axis_name="c", subcore_axis_name="s")
    out_type = [jax.ShapeDtypeStruct((NC, N_PAD, D_HID), jnp.float32)]
    if with_cnt:
        out_type.append(jax.ShapeDtypeStruct((NC, N_PAD), jnp.float32))

    scratch = [
        pltpu.VMEM((kmax, B_EDGE), jnp.int32),
        pltpu.VMEM((kmax, B_EDGE), jnp.int32),
        pltpu.VMEM((max(1, k1), B_EDGE), jnp.int32),
        pltpu.VMEM((RING, B_EDGE, D_HID), jnp.float32),
        pltpu.VMEM((rps, D_HID), jnp.float32),
        pltpu.VMEM((rps,), jnp.float32),
        pltpu.VMEM((B_EDGE,), jnp.float32),
        pltpu.VMEM_SHARED((N_PAD, D_HID), jnp.float32),
        pltpu.VMEM_SHARED((N_PAD,), jnp.float32),
        pltpu.SemaphoreType.DMA((RING,)),
        pltpu.SemaphoreType.DMA((RING,)),
        pltpu.SemaphoreType.DMA((RING,)),
    ]

    @pl.kernel(
        mesh=mesh,
        out_type=tuple(out_type),
        scratch_types=scratch,
        compiler_params=pltpu.CompilerParams(use_tc_tiling_on_sc=False),
    )
    def k(*refs):
        if with_cnt:
            (y_hbm, ei_hbm, out_hbm, cnt_hbm,
             src_v, dst_v, dstx_v, rows_v, zrow_v, zcnt_v, ones_v,
             acc_s, cnt_s, gsem, ssem, csem) = refs
        else:
            (y_hbm, ei_hbm, out_hbm,
             src_v, dst_v, dstx_v, rows_v, zrow_v, zcnt_v, ones_v,
             acc_s, cnt_s, gsem, ssem, csem) = refs
            cnt_hbm = None
        src_hbm = ei_hbm.at[0]
        dst_hbm = ei_hbm.at[1]

        c = lax.axis_index("c")
        s = lax.axis_index("s")
        row0 = s * rps

        def g_start(j, sl):
            pltpu.async_copy(y_hbm.at[src_v.at[j]], rows_v.at[sl],
                             gsem.at[sl])

        def g_wait(j, sl):
            pltpu.make_async_copy(y_hbm.at[src_v.at[j]], rows_v.at[sl],
                                  gsem.at[sl]).wait()

        def s_start(j, sl):
            pltpu.async_copy(rows_v.at[sl], acc_s.at[dst_v.at[j]],
                             ssem.at[sl], add=True)

        def s_wait(j, sl):
            pltpu.make_async_copy(rows_v.at[sl], acc_s.at[dst_v.at[j]],
                                  ssem.at[sl]).wait()

        def run(kc, base_chunk, do_cnt, ncx, base_cnt):
            def cref(j):
                return dst_v.at[j] if j < kc else dstx_v.at[j - kc]

            def c_start(j):
                pltpu.async_copy(ones_v, cnt_s.at[cref(j)],
                                 csem.at[j % RING], add=True)

            def c_wait(j):
                pltpu.make_async_copy(ones_v, cnt_s.at[cref(j)],
                                      csem.at[j % RING]).wait()

            if kc > 0:
                pltpu.sync_copy(src_hbm.at[pl.ds(base_chunk, kc)],
                                src_v.at[pl.ds(0, kc)])
                pltpu.sync_copy(dst_hbm.at[pl.ds(base_chunk, kc)],
                                dst_v.at[pl.ds(0, kc)])
                for j0 in range(min(LOOK, kc)):
                    g_start(j0, j0 % RING)
            if do_cnt and ncx > 0:
                pltpu.sync_copy(dst_hbm.at[pl.ds(base_cnt, ncx)],
                                dstx_v.at[pl.ds(0, ncx)])

            def zr(i, carry):
                zrow_v[i] = jnp.zeros((D_HID,), jnp.float32)
                return carry
            lax.fori_loop(0, rps, zr, 0)
            pltpu.sync_copy(zrow_v, acc_s.at[pl.ds(row0, rps)])
            if with_cnt:
                def zc(i, carry):
                    zcnt_v[pl.ds(i * 16, 16)] = jnp.zeros((16,), jnp.float32)
                    return carry
                lax.fori_loop(0, rps // 16, zc, 0)
                pltpu.sync_copy(zcnt_v, cnt_s.at[pl.ds(row0, rps)])

                if do_cnt:
                    def of(i, carry):
                        ones_v[pl.ds(i * 16, 16)] = jnp.ones((16,),
                                                             jnp.float32)
                        return carry
                    lax.fori_loop(0, B_EDGE // 16, of, 0)
            plsc.subcore_barrier()

            n_cnt = kc + ncx if do_cnt else 0
            for j in range(max(kc, n_cnt)):
                if j < kc:
                    sl = j % RING
                    jn = j + LOOK
                    if jn < kc:
                        if jn - RING >= 0:
                            s_wait(jn - RING, jn % RING)
                        g_start(jn, jn % RING)
                    g_wait(j, sl)
                    s_start(j, sl)
                if j < n_cnt:
                    if j - RING >= 0:
                        c_wait(j - RING)
                    c_start(j)
            for j in range(max(0, kc - RING), kc):
                s_wait(j, j % RING)
            for j in range(max(0, n_cnt - RING), n_cnt):
                c_wait(j)

        @pl.when(c == 0)
        def _():
            run(k0, s * k0, with_cnt, k1, NS * k0 + s * k1)

        @pl.when(c != 0)
        def _():
            run(k1, NS * k0 + s * k1, False, 0, 0)

        plsc.subcore_barrier()

        pltpu.sync_copy(acc_s.at[pl.ds(row0, rps)],
                        out_hbm.at[c, pl.ds(row0, rps)])
        if with_cnt:
            pltpu.sync_copy(cnt_s.at[pl.ds(row0, rps)],
                            cnt_hbm.at[c, pl.ds(row0, rps)])

    return k(y, ei3)


def _proj_body(x_ref, wl_ref, wr_ref, y_ref, r_ref):
    x = x_ref[...]
    y_ref[...] = jnp.dot(x, wl_ref[...], preferred_element_type=jnp.float32)
    r_ref[...] = jnp.dot(x, wr_ref[...], preferred_element_type=jnp.float32)


def _mid_body(s_ref, c_ref, r_ref, b_ref, h_ref, inv_ref):
    cnt = c_ref[0, :N_NODES] + c_ref[1, :N_NODES]
    inv = 1.0 / jnp.maximum(cnt, 1.0)
    agg = (s_ref[0, :N_NODES, :] + s_ref[1, :N_NODES, :]) * inv[:, None]
    t = agg + b_ref[...][None, :] + r_ref[...]
    h_ref[...] = jnp.where(t > 0, t, jnp.exp(t) - 1.0)
    inv_ref[...] = inv[:, None]


def _out_body(s_ref, h_ref, inv_ref, wl_ref, b_ref, wr_ref, o_ref):
    agg = (s_ref[0, :N_NODES, :] + s_ref[1, :N_NODES, :]) * inv_ref[...]
    h = h_ref[...]
    t = (jnp.dot(agg, wl_ref[...], preferred_element_type=jnp.float32)
         + b_ref[...][None, :]
         + jnp.dot(h, wr_ref[...], preferred_element_type=jnp.float32))
    m = jnp.max(t, axis=1, keepdims=True)
    lse = jnp.log(jnp.sum(jnp.exp(t - m), axis=1, keepdims=True)) + m
    o_ref[...] = t - lse


def kernel(x, edge_index, W_l1, b_l1, W_r1, W_l2, b_l2, W_r2):
    n, d_in = x.shape
    e = edge_index.shape[1]
    ch = _cdiv(_cdiv(e, B_EDGE), NS) * NS
    e_pad = ch * B_EDGE

    ei = edge_index
    if e_pad != e:
        pad = jnp.stack([jnp.zeros((e_pad - e,), ei.dtype),
                         jnp.full((e_pad - e,), N_NODES, ei.dtype)])
        ei = jnp.concatenate([ei, pad], axis=1)
    ei3 = ei.reshape(2, ch, B_EDGE)

    y1, r1 = pl.pallas_call(
        _proj_body,
        out_shape=[jax.ShapeDtypeStruct((n, D_HID), jnp.float32)] * 2,
    )(x, W_l1, W_r1)

    psum1, pcnt1 = _seg_sum(y1, ei3, with_cnt=True)

    h, inv = pl.pallas_call(
        _mid_body,
        out_shape=[
            jax.ShapeDtypeStruct((n, D_HID), jnp.float32),
            jax.ShapeDtypeStruct((n, 1), jnp.float32),
        ],
    )(psum1, pcnt1, r1, b_l1)

    (psum2,) = _seg_sum(h, ei3, with_cnt=False)

    out = pl.pallas_call(
        _out_body,
        out_shape=jax.ShapeDtypeStruct((n, W_l2.shape[1]), jnp.float32),
    )(psum2, h, inv, W_l2, b_l2, W_r2)
    return out


# speedup vs baseline: 1.0233x; 1.0233x over previous
import jax
import jax.numpy as jnp
from jax import lax
from jax.experimental import pallas as pl
from jax.experimental.pallas import tpu as pltpu
from jax.experimental.pallas import tpu_sc as plsc

N_NODES = 10000
D_HID = 16

NC = 2
NS = 16
NW = NC * NS
B_EDGE = 800
N_PAD = 10240
FRAC0 = 0.88
RING = 4
LOOK = 2


def _cdiv(a, b):
    return (a + b - 1) // b


def _seg_sum(y, ei3, with_cnt):
    ch = ei3.shape[1]
    cht = ch // NS
    k0 = max(1, min(cht, round(cht * FRAC0)))
    k1 = cht - k0
    kmax = max(k0, k1)
    rps = N_PAD // NS

    mesh = plsc.VectorSubcoreMesh(core_axis_name="c", subcore_axis_name="s")
    out_type = [jax.ShapeDtypeStruct((NC, N_PAD, D_HID), jnp.float32)]
    if with_cnt:
        out_type.append(jax.ShapeDtypeStruct((NC, N_PAD), jnp.float32))

    scratch = [
        pltpu.VMEM((kmax, B_EDGE), jnp.int32),
        pltpu.VMEM((kmax, B_EDGE), jnp.int32),
        pltpu.VMEM((max(1, k1), B_EDGE), jnp.int32),
        pltpu.VMEM((RING, B_EDGE, D_HID), jnp.float32),
        pltpu.VMEM((rps, D_HID), jnp.float32),
        pltpu.VMEM((rps,), jnp.float32),
        pltpu.VMEM((B_EDGE,), jnp.float32),
        pltpu.VMEM_SHARED((N_PAD, D_HID), jnp.float32),
        pltpu.VMEM_SHARED((N_PAD,), jnp.float32),
        pltpu.SemaphoreType.DMA((RING,)),
        pltpu.SemaphoreType.DMA((RING,)),
        pltpu.SemaphoreType.DMA((RING,)),
    ]

    @pl.kernel(
        mesh=mesh,
        out_type=tuple(out_type),
        scratch_types=scratch,
        compiler_params=pltpu.CompilerParams(use_tc_tiling_on_sc=False),
    )
    def k(*refs):
        if with_cnt:
            (y_hbm, ei_hbm, out_hbm, cnt_hbm,
             src_v, dst_v, dstx_v, rows_v, zrow_v, zcnt_v, ones_v,
             acc_s, cnt_s, gsem, ssem, csem) = refs
        else:
            (y_hbm, ei_hbm, out_hbm,
             src_v, dst_v, dstx_v, rows_v, zrow_v, zcnt_v, ones_v,
             acc_s, cnt_s, gsem, ssem, csem) = refs
            cnt_hbm = None
        src_hbm = ei_hbm.at[0]
        dst_hbm = ei_hbm.at[1]

        c = lax.axis_index("c")
        s = lax.axis_index("s")
        row0 = s * rps

        def g_start(j, sl):
            pltpu.async_copy(y_hbm.at[src_v.at[j]], rows_v.at[sl],
                             gsem.at[sl])

        def g_wait(j, sl):
            pltpu.make_async_copy(y_hbm.at[src_v.at[j]], rows_v.at[sl],
                                  gsem.at[sl]).wait()

        def s_start(j, sl):
            pltpu.async_copy(rows_v.at[sl], acc_s.at[dst_v.at[j]],
                             ssem.at[sl], add=True)

        def s_wait(j, sl):
            pltpu.make_async_copy(rows_v.at[sl], acc_s.at[dst_v.at[j]],
                                  ssem.at[sl]).wait()

        def run(kc, base_chunk, do_cnt, ncx, base_cnt):
            def cref(j):
                return dst_v.at[j] if j < kc else dstx_v.at[j - kc]

            def c_start(j):
                pltpu.async_copy(ones_v, cnt_s.at[cref(j)],
                                 csem.at[j % RING], add=True)

            def c_wait(j):
                pltpu.make_async_copy(ones_v, cnt_s.at[cref(j)],
                                      csem.at[j % RING]).wait()

            if kc > 0:
                pltpu.sync_copy(src_hbm.at[pl.ds(base_chunk, kc)],
                                src_v.at[pl.ds(0, kc)])
                pltpu.sync_copy(dst_hbm.at[pl.ds(base_chunk, kc)],
                                dst_v.at[pl.ds(0, kc)])
                for j0 in range(min(LOOK, kc)):
                    g_start(j0, j0 % RING)
            if do_cnt and ncx > 0:
                pltpu.sync_copy(dst_hbm.at[pl.ds(base_cnt, ncx)],
                                dstx_v.at[pl.ds(0, ncx)])

            def zr(i, carry):
                zrow_v[i] = jnp.zeros((D_HID,), jnp.float32)
                return carry
            lax.fori_loop(0, rps, zr, 0)
            pltpu.sync_copy(zrow_v, acc_s.at[pl.ds(row0, rps)])
            if with_cnt:
                def zc(i, carry):
                    zcnt_v[pl.ds(i * 16, 16)] = jnp.zeros((16,), jnp.float32)
                    return carry
                lax.fori_loop(0, rps // 16, zc, 0)
                pltpu.sync_copy(zcnt_v, cnt_s.at[pl.ds(row0, rps)])

                if do_cnt:
                    def of(i, carry):
                        ones_v[pl.ds(i * 16, 16)] = jnp.ones((16,),
                                                             jnp.float32)
                        return carry
                    lax.fori_loop(0, B_EDGE // 16, of, 0)
            plsc.subcore_barrier()

            n_cnt = kc + ncx if do_cnt else 0
            for j in range(max(kc, n_cnt)):
                if j < kc:
                    sl = j % RING
                    jn = j + LOOK
                    if jn < kc:
                        if jn - RING >= 0:
                            s_wait(jn - RING, jn % RING)
                        g_start(jn, jn % RING)
                    g_wait(j, sl)
                    s_start(j, sl)
                if j < n_cnt:
                    if j - RING >= 0:
                        c_wait(j - RING)
                    c_start(j)
            for j in range(max(0, kc - RING), kc):
                s_wait(j, j % RING)
            for j in range(max(0, n_cnt - RING), n_cnt):
                c_wait(j)

        @pl.when(c == 0)
        def _():
            run(k0, s * k0, with_cnt, k1, NS * k0 + s * k1)

        @pl.when(c != 0)
        def _():
            run(k1, NS * k0 + s * k1, False, 0, 0)

        plsc.subcore_barrier()

        pltpu.sync_copy(acc_s.at[pl.ds(row0, rps)],
                        out_hbm.at[c, pl.ds(row0, rps)])
        if with_cnt:
            pltpu.sync_copy(cnt_s.at[pl.ds(row0, rps)],
                            cnt_hbm.at[c, pl.ds(row0, rps)])

    return k(y, ei3)


def _proj_body(x_ref, wl_ref, wr_ref, y_ref, r_ref):
    x = x_ref[...]
    y_ref[...] = jnp.dot(x, wl_ref[...], preferred_element_type=jnp.float32)
    r_ref[...] = jnp.dot(x, wr_ref[...], preferred_element_type=jnp.float32)


def _mid_body(s_ref, c_ref, r_ref, b_ref, h_ref, inv_ref):
    cnt = c_ref[0, :N_NODES] + c_ref[1, :N_NODES]
    inv = 1.0 / jnp.maximum(cnt, 1.0)
    agg = (s_ref[0, :N_NODES, :] + s_ref[1, :N_NODES, :]) * inv[:, None]
    t = agg + b_ref[...][None, :] + r_ref[...]
    h_ref[...] = jnp.where(t > 0, t, jnp.exp(t) - 1.0)
    inv_ref[...] = inv[:, None]


def _out_body(s_ref, h_ref, inv_ref, wl_ref, b_ref, wr_ref, o_ref):
    agg = (s_ref[0, :N_NODES, :] + s_ref[1, :N_NODES, :]) * inv_ref[...]
    h = h_ref[...]
    t = (jnp.dot(agg, wl_ref[...], preferred_element_type=jnp.float32)
         + b_ref[...][None, :]
         + jnp.dot(h, wr_ref[...], preferred_element_type=jnp.float32))
    m = jnp.max(t, axis=1, keepdims=True)
    lse = jnp.log(jnp.sum(jnp.exp(t - m), axis=1, keepdims=True)) + m
    o_ref[...] = t - lse


def kernel(x, edge_index, W_l1, b_l1, W_r1, W_l2, b_l2, W_r2):
    n, d_in = x.shape
    e = edge_index.shape[1]
    ch = _cdiv(_cdiv(e, B_EDGE), NS) * NS
    e_pad = ch * B_EDGE

    ei = edge_index
    if e_pad != e:
        pad = jnp.stack([jnp.zeros((e_pad - e,), ei.dtype),
                         jnp.full((e_pad - e,), N_NODES, ei.dtype)])
        ei = jnp.concatenate([ei, pad], axis=1)
    ei3 = ei.reshape(2, ch, B_EDGE)

    y1, r1 = pl.pallas_call(
        _proj_body,
        out_shape=[jax.ShapeDtypeStruct((n, D_HID), jnp.float32)] * 2,
    )(x, W_l1, W_r1)

    psum1, pcnt1 = _seg_sum(y1, ei3, with_cnt=True)

    h, inv = pl.pallas_call(
        _mid_body,
        out_shape=[
            jax.ShapeDtypeStruct((n, D_HID), jnp.float32),
            jax.ShapeDtypeStruct((n, 1), jnp.float32),
        ],
    )(psum1, pcnt1, r1, b_l1)

    (psum2,) = _seg_sum(h, ei3, with_cnt=False)

    out = pl.pallas_call(
        _out_body,
        out_shape=jax.ShapeDtypeStruct((n, W_l2.shape[1]), jnp.float32),
    )(psum2, h, inv, W_l2, b_l2, W_r2)
    return out


# speedup vs baseline: 1.0480x; 1.0241x over previous
import jax
import jax.numpy as jnp
from jax import lax
from jax.experimental import pallas as pl
from jax.experimental.pallas import tpu as pltpu
from jax.experimental.pallas import tpu_sc as plsc

N_NODES = 10000
D_HID = 16

NC = 2
NS = 16
NW = NC * NS
B_EDGE = 800
N_PAD = 10240
FRAC0 = 0.8
RING = 4
LOOK = 2


def _cdiv(a, b):
    return (a + b - 1) // b


def _seg_sum(y, ei3, with_cnt):
    ch = ei3.shape[1]
    cht = ch // NS
    k0 = max(1, min(cht, round(cht * FRAC0)))
    k1 = cht - k0
    kmax = max(k0, k1)
    rps = N_PAD // NS

    mesh = plsc.VectorSubcoreMesh(core_axis_name="c", subcore_axis_name="s")
    out_type = [jax.ShapeDtypeStruct((NC, N_PAD, D_HID), jnp.float32)]
    if with_cnt:
        out_type.append(jax.ShapeDtypeStruct((NC, N_PAD), jnp.float32))

    scratch = [
        pltpu.VMEM((kmax, B_EDGE), jnp.int32),
        pltpu.VMEM((kmax, B_EDGE), jnp.int32),
        pltpu.VMEM((max(1, k1), B_EDGE), jnp.int32),
        pltpu.VMEM((RING, B_EDGE, D_HID), jnp.float32),
        pltpu.VMEM((rps, D_HID), jnp.float32),
        pltpu.VMEM((rps,), jnp.float32),
        pltpu.VMEM((B_EDGE,), jnp.float32),
        pltpu.VMEM_SHARED((N_PAD, D_HID), jnp.float32),
        pltpu.VMEM_SHARED((N_PAD,), jnp.float32),
        pltpu.SemaphoreType.DMA((RING,)),
        pltpu.SemaphoreType.DMA((RING,)),
        pltpu.SemaphoreType.DMA((RING,)),
    ]

    @pl.kernel(
        mesh=mesh,
        out_type=tuple(out_type),
        scratch_types=scratch,
        compiler_params=pltpu.CompilerParams(use_tc_tiling_on_sc=False),
    )
    def k(*refs):
        if with_cnt:
            (y_hbm, ei_hbm, out_hbm, cnt_hbm,
             src_v, dst_v, dstx_v, rows_v, zrow_v, zcnt_v, ones_v,
             acc_s, cnt_s, gsem, ssem, csem) = refs
        else:
            (y_hbm, ei_hbm, out_hbm,
             src_v, dst_v, dstx_v, rows_v, zrow_v, zcnt_v, ones_v,
             acc_s, cnt_s, gsem, ssem, csem) = refs
            cnt_hbm = None
        src_hbm = ei_hbm.at[0]
        dst_hbm = ei_hbm.at[1]

        c = lax.axis_index("c")
        s = lax.axis_index("s")
        row0 = s * rps

        def g_start(j, sl):
            pltpu.async_copy(y_hbm.at[src_v.at[j]], rows_v.at[sl],
                             gsem.at[sl])

        def g_wait(j, sl):
            pltpu.make_async_copy(y_hbm.at[src_v.at[j]], rows_v.at[sl],
                                  gsem.at[sl]).wait()

        def s_start(j, sl):
            pltpu.async_copy(rows_v.at[sl], acc_s.at[dst_v.at[j]],
                             ssem.at[sl], add=True)

        def s_wait(j, sl):
            pltpu.make_async_copy(rows_v.at[sl], acc_s.at[dst_v.at[j]],
                                  ssem.at[sl]).wait()

        def run(kc, base_chunk, do_cnt, ncx, base_cnt):
            def cref(j):
                return dst_v.at[j] if j < kc else dstx_v.at[j - kc]

            def c_start(j):
                pltpu.async_copy(ones_v, cnt_s.at[cref(j)],
                                 csem.at[j % RING], add=True)

            def c_wait(j):
                pltpu.make_async_copy(ones_v, cnt_s.at[cref(j)],
                                      csem.at[j % RING]).wait()

            if kc > 0:
                pltpu.sync_copy(src_hbm.at[pl.ds(base_chunk, kc)],
                                src_v.at[pl.ds(0, kc)])
                pltpu.sync_copy(dst_hbm.at[pl.ds(base_chunk, kc)],
                                dst_v.at[pl.ds(0, kc)])
                for j0 in range(min(LOOK, kc)):
                    g_start(j0, j0 % RING)
            if do_cnt and ncx > 0:
                pltpu.sync_copy(dst_hbm.at[pl.ds(base_cnt, ncx)],
                                dstx_v.at[pl.ds(0, ncx)])

            def zr(i, carry):
                zrow_v[i] = jnp.zeros((D_HID,), jnp.float32)
                return carry
            lax.fori_loop(0, rps, zr, 0)
            pltpu.sync_copy(zrow_v, acc_s.at[pl.ds(row0, rps)])
            if with_cnt:
                def zc(i, carry):
                    zcnt_v[pl.ds(i * 16, 16)] = jnp.zeros((16,), jnp.float32)
                    return carry
                lax.fori_loop(0, rps // 16, zc, 0)
                pltpu.sync_copy(zcnt_v, cnt_s.at[pl.ds(row0, rps)])

                if do_cnt:
                    def of(i, carry):
                        ones_v[pl.ds(i * 16, 16)] = jnp.ones((16,),
                                                             jnp.float32)
                        return carry
                    lax.fori_loop(0, B_EDGE // 16, of, 0)
            plsc.subcore_barrier()

            n_cnt = kc + ncx if do_cnt else 0
            for j in range(max(kc, n_cnt)):
                if j < kc:
                    sl = j % RING
                    jn = j + LOOK
                    if jn < kc:
                        if jn - RING >= 0:
                            s_wait(jn - RING, jn % RING)
                        g_start(jn, jn % RING)
                    g_wait(j, sl)
                    s_start(j, sl)
                if j < n_cnt:
                    if j - RING >= 0:
                        c_wait(j - RING)
                    c_start(j)
            for j in range(max(0, kc - RING), kc):
                s_wait(j, j % RING)
            for j in range(max(0, n_cnt - RING), n_cnt):
                c_wait(j)

        @pl.when(c == 0)
        def _():
            run(k0, s * k0, with_cnt, k1, NS * k0 + s * k1)

        @pl.when(c != 0)
        def _():
            run(k1, NS * k0 + s * k1, False, 0, 0)

        plsc.subcore_barrier()

        pltpu.sync_copy(acc_s.at[pl.ds(row0, rps)],
                        out_hbm.at[c, pl.ds(row0, rps)])
        if with_cnt:
            pltpu.sync_copy(cnt_s.at[pl.ds(row0, rps)],
                            cnt_hbm.at[c, pl.ds(row0, rps)])

    return k(y, ei3)


def _proj_body(x_ref, wl_ref, wr_ref, y_ref, r_ref):
    x = x_ref[...]
    y_ref[...] = jnp.dot(x, wl_ref[...], preferred_element_type=jnp.float32)
    r_ref[...] = jnp.dot(x, wr_ref[...], preferred_element_type=jnp.float32)


def _mid_body(s_ref, c_ref, r_ref, b_ref, h_ref, inv_ref):
    cnt = c_ref[0, :N_NODES] + c_ref[1, :N_NODES]
    inv = 1.0 / jnp.maximum(cnt, 1.0)
    agg = (s_ref[0, :N_NODES, :] + s_ref[1, :N_NODES, :]) * inv[:, None]
    t = agg + b_ref[...][None, :] + r_ref[...]
    h_ref[...] = jnp.where(t > 0, t, jnp.exp(t) - 1.0)
    inv_ref[...] = inv[:, None]


def _out_body(s_ref, h_ref, inv_ref, wl_ref, b_ref, wr_ref, o_ref):
    agg = (s_ref[0, :N_NODES, :] + s_ref[1, :N_NODES, :]) * inv_ref[...]
    h = h_ref[...]
    t = (jnp.dot(agg, wl_ref[...], preferred_element_type=jnp.float32)
         + b_ref[...][None, :]
         + jnp.dot(h, wr_ref[...], preferred_element_type=jnp.float32))
    m = jnp.max(t, axis=1, keepdims=True)
    lse = jnp.log(jnp.sum(jnp.exp(t - m), axis=1, keepdims=True)) + m
    o_ref[...] = t - lse


def kernel(x, edge_index, W_l1, b_l1, W_r1, W_l2, b_l2, W_r2):
    n, d_in = x.shape
    e = edge_index.shape[1]
    ch = _cdiv(_cdiv(e, B_EDGE), NS) * NS
    e_pad = ch * B_EDGE

    ei = edge_index
    if e_pad != e:
        pad = jnp.stack([jnp.zeros((e_pad - e,), ei.dtype),
                         jnp.full((e_pad - e,), N_NODES, ei.dtype)])
        ei = jnp.concatenate([ei, pad], axis=1)
    ei3 = ei.reshape(2, ch, B_EDGE)

    y1, r1 = pl.pallas_call(
        _proj_body,
        out_shape=[jax.ShapeDtypeStruct((n, D_HID), jnp.float32)] * 2,
    )(x, W_l1, W_r1)

    psum1, pcnt1 = _seg_sum(y1, ei3, with_cnt=True)

    h, inv = pl.pallas_call(
        _mid_body,
        out_shape=[
            jax.ShapeDtypeStruct((n, D_HID), jnp.float32),
            jax.ShapeDtypeStruct((n, 1), jnp.float32),
        ],
    )(psum1, pcnt1, r1, b_l1)

    (psum2,) = _seg_sum(h, ei3, with_cnt=False)

    out = pl.pallas_call(
        _out_body,
        out_shape=jax.ShapeDtypeStruct((n, W_l2.shape[1]), jnp.float32),
    )(psum2, h, inv, W_l2, b_l2, W_r2)
    return out
